# baseline (speedup 1.0000x reference)
import functools

import jax
import jax.numpy as jnp
from jax import lax
from jax.experimental import pallas as pl
from jax.experimental.pallas import tpu as pltpu
from jax.experimental.pallas import tpu_sc as plsc

_DIM = 32768
_K = 4096
_BATCH = 128
_ROW_BLK = 64

_NSUB = 16
_PER_TILE = _DIM // _NSUB
_NVEC = _PER_TILE // 16


def _sc_threshold_body(wbits_hbm, out_hbm, wbuf, keybuf, hist, merge, tref,
                       outstage, shist):
    cid = lax.axis_index("c")
    sid = lax.axis_index("s")
    base = sid * _PER_TILE

    pltpu.sync_copy(wbits_hbm.at[pl.ds(base, _PER_TILE)], wbuf)

    def to_key(i, carry):
        bits = wbuf[pl.ds(i * 16, 16)]
        neg = bits >= jnp.uint32(0x80000000)
        key = jnp.where(neg, ~bits, bits | jnp.uint32(0x80000000))
        keybuf[pl.ds(i * 16, 16)] = key
        return carry

    lax.fori_loop(0, _NVEC, to_key, 0)

    zeros16 = jnp.zeros((16,), jnp.int32)
    ones16 = jnp.full((16,), 1, jnp.int32)
    iota16 = lax.iota(jnp.int32, 16)
    tref[...] = jnp.zeros((16,), jnp.uint32)

    def round_body(r, carry):
        b = (jnp.int32(28) - jnp.int32(4) * r).astype(jnp.uint32)
        t = tref[...]
        hist[pl.ds(0, 16)] = zeros16
        hist[pl.ds(16, 16)] = zeros16

        def count_body(i, c):
            key = keybuf[pl.ds(i * 16, 16)]
            valid = key >= t
            d = jnp.minimum((key - t) >> b, jnp.uint32(16)).astype(jnp.int32)
            plsc.addupdate_scatter(hist, [d], ones16, mask=valid)
            return c

        lax.fori_loop(0, _NVEC, count_body, 0)

        pltpu.sync_copy(hist, shist.at[pl.ds(sid * 32, 32)])
        plsc.subcore_barrier()
        pltpu.sync_copy(shist, merge)
        plsc.subcore_barrier()

        acc_lo = zeros16
        acc_hi = zeros16
        for row in range(_NSUB):
            acc_lo = acc_lo + merge[pl.ds(row * 32, 16)]
            acc_hi = acc_hi + merge[pl.ds(row * 32 + 16, 16)]
        over = jnp.sum(jnp.where(iota16 == 0, acc_hi, zeros16))
        sfx = lax.rev(jnp.cumsum(lax.rev(acc_lo, (0,))), (0,))
        counts = sfx + over
        sat = counts >= _K
        d_star = plsc.all_reduce_population_count(sat) - 1
        tref[...] = t | (d_star.astype(jnp.uint32) << b)
        return carry

    lax.fori_loop(0, 8, round_body, 0)

    @pl.when(jnp.logical_and(cid == 0, sid == 0))
    def _write():
        t = tref[...]
        for i in range(8):
            outstage[pl.ds(i * 16, 16)] = t
        pltpu.sync_copy(outstage, out_hbm)


def _sc_threshold(w):
    mesh = plsc.VectorSubcoreMesh(core_axis_name="c", subcore_axis_name="s")
    kern = functools.partial(
        pl.kernel,
        mesh=mesh,
        compiler_params=pltpu.CompilerParams(needs_layout_passes=False),
        out_type=jax.ShapeDtypeStruct((128,), jnp.uint32),
        scratch_types=[
            pltpu.VMEM((_PER_TILE,), jnp.uint32),
            pltpu.VMEM((_PER_TILE,), jnp.uint32),
            pltpu.VMEM((32,), jnp.int32),
            pltpu.VMEM((_NSUB * 32,), jnp.int32),
            pltpu.VMEM((16,), jnp.uint32),
            pltpu.VMEM((128,), jnp.uint32),
            pltpu.VMEM_SHARED((_NSUB * 32,), jnp.int32),
        ],
    )(_sc_threshold_body)
    return kern(w)


def _tc_mask_body(thr_ref, w_ref, x_ref, o_ref, mask_ref):
    @pl.when(pl.program_id(0) == 0)
    def _compute_mask():
        bits = lax.bitcast_convert_type(w_ref[...], jnp.uint32)
        neg = bits >= jnp.uint32(0x80000000)
        key = jnp.where(neg, ~bits, bits | jnp.uint32(0x80000000))
        mask_ref[...] = (key >= thr_ref[0]).astype(jnp.float32)

    o_ref[...] = x_ref[...] * mask_ref[...]


def kernel(inputs, w):
    wbits = lax.bitcast_convert_type(w, jnp.uint32)
    thr = _sc_threshold(wbits)
    w2 = w.reshape(1, _DIM)
    return pl.pallas_call(
        _tc_mask_body,
        grid=(_BATCH // _ROW_BLK,),
        in_specs=[
            pl.BlockSpec(memory_space=pltpu.SMEM),
            pl.BlockSpec((1, _DIM), lambda i: (0, 0)),
            pl.BlockSpec((_ROW_BLK, _DIM), lambda i: (i, 0)),
        ],
        out_specs=pl.BlockSpec((_ROW_BLK, _DIM), lambda i: (i, 0)),
        out_shape=jax.ShapeDtypeStruct((_BATCH, _DIM), jnp.float32),
        scratch_shapes=[pltpu.VMEM((1, _DIM), jnp.float32)],
    )(thr[0:1], w2, inputs)


# speedup vs baseline: 3.2036x; 3.2036x over previous
import jax
import jax.numpy as jnp
from jax import lax
from jax.experimental import pallas as pl
from jax.experimental.pallas import tpu as pltpu

_DIM = 32768
_K = 4096
_BATCH = 128
_ROW_BLK = 64


def _body(w_ref, x_ref, o_ref, mask_ref):
    @pl.when(pl.program_id(0) == 0)
    def _compute_mask():
        w = w_ref[...]
        bits = lax.bitcast_convert_type(w, jnp.uint32)
        neg = bits >= jnp.uint32(0x80000000)
        key = jnp.where(neg, ~bits, bits | jnp.uint32(0x80000000))

        jvec = lax.broadcasted_iota(jnp.uint32, (16, 1), 0)

        def step(i, t):
            b = jnp.uint32(28) - jnp.uint32(4) * i.astype(jnp.uint32)
            cands = t | jnp.left_shift(jvec, b)
            cnts = jnp.sum((key >= cands).astype(jnp.int32), axis=1,
                           keepdims=True)
            j_star = jnp.sum((cnts >= _K).astype(jnp.int32), axis=0,
                             keepdims=True) - 1
            return t | jnp.left_shift(
                jnp.broadcast_to(j_star.astype(jnp.uint32), (16, 1)), b)

        t = lax.fori_loop(0, 8, step, jnp.zeros((16, 1), jnp.uint32))
        mask_ref[...] = (key >= t[0:1, 0:1]).astype(jnp.float32)

    o_ref[...] = x_ref[...] * mask_ref[...]


def kernel(inputs, w):
    w2 = w.reshape(1, _DIM)
    return pl.pallas_call(
        _body,
        grid=(_BATCH // _ROW_BLK,),
        in_specs=[
            pl.BlockSpec((1, _DIM), lambda i: (0, 0)),
            pl.BlockSpec((_ROW_BLK, _DIM), lambda i: (i, 0)),
        ],
        out_specs=pl.BlockSpec((_ROW_BLK, _DIM), lambda i: (i, 0)),
        out_shape=jax.ShapeDtypeStruct((_BATCH, _DIM), jnp.float32),
        scratch_shapes=[pltpu.VMEM((1, _DIM), jnp.float32)],
    )(w2, inputs)
